# Initial kernel scaffold; baseline (speedup 1.0000x reference)
import jax
import jax.numpy as jnp
from jax.experimental import pallas as pl


def kernel(x_real, x_imag, index, out_shape):
    raise NotImplementedError("write your pallas kernel here")


# speedup vs baseline: 1.0390x; 1.0390x over previous
import jax
import jax.numpy as jnp
from jax import lax
from jax.experimental import pallas as pl
from jax.experimental.pallas import tpu as pltpu
from jax.experimental.pallas import tpu_sc as plsc

B, C, H, W = 4, 96, 112, 112
OH, OW = 224, 224
N = B * C
HW = H * W
S = OH * OW
LANES = 16
NC, NS = 2, 16
NWORKERS = NC * NS
PAIRS_PER_W = N // NWORKERS
CH = 6272
NCHUNK = S // CH
VEC_PER_CHUNK = CH // LANES

_EPS = 1e-8


def _rsqrt(s):
    i = jnp.int32(0x5F3759DF) - (lax.bitcast_convert_type(s, jnp.int32) >> 1)
    z = lax.bitcast_convert_type(i, jnp.float32)
    for _ in range(3):
        z = z * (1.5 - 0.5 * s * z * z)
    return z


def _div7(u):
    return (u * 9363) >> 16


def _sc_body(x_hbm, y_hbm, mag_hbm, outr_hbm, outi_hbm,
             ur_v, ui_v, mag_v, outr_v, outi_v):
    wid = lax.axis_index("s") * NC + lax.axis_index("c")
    lane = lax.iota(jnp.int32, LANES)

    def pair_body(k, _):
        pr = wid * PAIRS_PER_W + k
        pltpu.sync_copy(x_hbm.at[pl.ds(pr * HW, HW)], ur_v)
        pltpu.sync_copy(y_hbm.at[pl.ds(pr * HW, HW)], ui_v)

        def phasor_body(i, _):
            ds = pl.ds(i * LANES, LANES)
            x = ur_v[ds]
            y = ui_v[ds]
            yc = jnp.where((y < _EPS) & (y > -_EPS), _EPS, y)
            z = _rsqrt(x * x + yc * yc)
            ur_v[ds] = x * z
            ui_v[ds] = yc * z
            return ()

        lax.fori_loop(0, HW // LANES, phasor_body, ())

        def chunk_body(cc, _):
            cbase = pr * S + cc * CH
            pltpu.sync_copy(mag_hbm.at[pl.ds(cbase, CH)], mag_v)

            def vec_body(j, _):
                base = j * LANES
                t = cc * CH + base + lane
                oh2 = _div7(t >> 6)
                v = t >> 1
                vm = v - 112 * _div7(v >> 4)
                p = oh2 * 112 + vm
                gr = plsc.load_gather(ur_v, [p])
                gi = plsc.load_gather(ui_v, [p])
                mg = mag_v[pl.ds(base, LANES)]
                outr_v[pl.ds(base, LANES)] = mg * gr
                outi_v[pl.ds(base, LANES)] = mg * gi
                return ()

            lax.fori_loop(0, VEC_PER_CHUNK, vec_body, ())
            pltpu.sync_copy(outr_v, outr_hbm.at[pl.ds(cbase, CH)])
            pltpu.sync_copy(outi_v, outi_hbm.at[pl.ds(cbase, CH)])
            return ()

        lax.fori_loop(0, NCHUNK, chunk_body, ())
        return ()

    lax.fori_loop(0, PAIRS_PER_W, pair_body, ())


@jax.jit
def _phase_mul(x, y, mag):
    f = pl.kernel(
        _sc_body,
        out_type=[
            jax.ShapeDtypeStruct((N * S,), jnp.float32),
            jax.ShapeDtypeStruct((N * S,), jnp.float32),
        ],
        mesh=plsc.VectorSubcoreMesh(core_axis_name="c", subcore_axis_name="s"),
        compiler_params=pltpu.CompilerParams(needs_layout_passes=False),
        scratch_types=[
            pltpu.VMEM((HW,), jnp.float32),
            pltpu.VMEM((HW,), jnp.float32),
            pltpu.VMEM((CH,), jnp.float32),
            pltpu.VMEM((CH,), jnp.float32),
            pltpu.VMEM((CH,), jnp.float32),
        ],
    )
    return f(x, y, mag)


def kernel(x_real, x_imag, index, out_shape):
    m = jnp.sqrt(x_real * x_real + x_imag * x_imag).reshape(B, C, HW)
    idx = index.reshape(B, C, HW)
    bi = jnp.arange(B)[:, None, None]
    ci = jnp.arange(C)[None, :, None]
    mag = jnp.zeros((B, C, S), dtype=jnp.float32).at[bi, ci, idx].set(m)

    outr, outi = _phase_mul(
        x_real.reshape(N * HW), x_imag.reshape(N * HW), mag.reshape(N * S)
    )
    return lax.complex(outr, outi).reshape(B, C, OH, OW)
